# Initial kernel scaffold; baseline (speedup 1.0000x reference)
import jax
import jax.numpy as jnp
from jax.experimental import pallas as pl


def kernel(inputs, W):
    raise NotImplementedError("write your pallas kernel here")


# speedup vs baseline: 136.3504x; 136.3504x over previous
import functools

import jax
import jax.numpy as jnp
from jax import lax
from jax.experimental import pallas as pl
from jax.experimental.pallas import tpu as pltpu
from jax.experimental.pallas import tpu_sc as plsc

B = 4096
L = 200
V = 100000
D = 3
NUM_TABLES = 10
T_LIST = (0, 1, 2, 3, 4, 6, 7, 8, 9)
NT = len(T_LIST)
CW = 32
NC = 2
NS = 16
NW = NC * NS
CHUNK = 100
ROWS_PER_W = B // NW
CHUNKS_PER_W = ROWS_PER_W * L // CHUNK


def _sc_body(wcat_hbm, idx_hbm, lidx_hbm, zeros_hbm, out_hbm,
             idx_v, lidx_v, rows0, rows1, acc, sem0, sem1):
    c = lax.axis_index("c")
    s = lax.axis_index("s")
    w = c * NS + s

    pltpu.sync_copy(idx_hbm.at[pl.ds(w * CHUNKS_PER_W, CHUNKS_PER_W)], idx_v)
    pltpu.sync_copy(lidx_hbm, lidx_v)

    @pl.when(s == 0)
    def _init():
        pltpu.sync_copy(zeros_hbm, acc)

    plsc.subcore_barrier()

    rows = (rows0, rows1)
    sems = (sem0, sem1)

    pltpu.async_copy(wcat_hbm.at[idx_v.at[0]], rows0, sem0)
    pltpu.async_copy(wcat_hbm.at[idx_v.at[1]], rows1, sem1)

    def body(i, carry):
        for par in range(2):
            j = i * 2 + par
            pltpu.make_async_copy(wcat_hbm.at[idx_v.at[0]], rows[par],
                                  sems[par]).wait()
            pltpu.sync_copy(rows[par], acc.at[lidx_v.at[par]], add=True)

            @pl.when(j + 2 < CHUNKS_PER_W)
            def _next():
                pltpu.async_copy(wcat_hbm.at[idx_v.at[j + 2]], rows[par],
                                 sems[par])
        return carry

    lax.fori_loop(0, CHUNKS_PER_W // 2, body, 0)

    plsc.subcore_barrier()

    @pl.when(s == 0)
    def _flush():
        pltpu.sync_copy(acc, out_hbm.at[c])


@jax.jit
def _sc_embed_sum(wcat, idx2, lidx, zeros):
    mesh = plsc.VectorSubcoreMesh(core_axis_name="c", subcore_axis_name="s")
    f = pl.kernel(
        _sc_body,
        out_type=jax.ShapeDtypeStruct((NC, L, CW), jnp.float32),
        mesh=mesh,
        compiler_params=pltpu.CompilerParams(use_tc_tiling_on_sc=False),
        scratch_types=[
            pltpu.VMEM((CHUNKS_PER_W, CHUNK), jnp.int32),
            pltpu.VMEM((2, CHUNK), jnp.int32),
            pltpu.VMEM((CHUNK, CW), jnp.float32),
            pltpu.VMEM((CHUNK, CW), jnp.float32),
            pltpu.VMEM_SHARED((L, CW), jnp.float32),
            pltpu.SemaphoreType.DMA,
            pltpu.SemaphoreType.DMA,
        ],
    )
    return f(wcat, idx2, lidx, zeros)


def kernel(inputs, W):
    wsel = W[jnp.array(T_LIST)]
    wcat = jnp.transpose(wsel, (1, 0, 2)).reshape(V, NT * D)
    wcat = jnp.pad(wcat, ((0, 0), (0, CW - NT * D)))

    idx2 = inputs.astype(jnp.int32).reshape(NW * CHUNKS_PER_W, CHUNK)
    lidx = jnp.arange(L, dtype=jnp.int32).reshape(2, CHUNK)
    zeros = jnp.zeros((L, CW), jnp.float32)

    parts = _sc_embed_sum(wcat, idx2, lidx, zeros)
    per_l = parts[0] + parts[1]

    g = jnp.transpose(per_l[:, : NT * D].reshape(L, NT, D), (1, 0, 2))
    row3 = jnp.sum(g[3], axis=0, keepdims=True)
    out = jnp.concatenate(
        [g[0], g[1], g[2], row3, g[4], row3, g[5], g[6], g[7], g[8]], axis=0
    )
    return out
